# baseline (speedup 1.0000x reference)
import math

import jax
import jax.numpy as jnp
from jax.experimental import pallas as pl
import jax.experimental.pallas.tpu as pltpu

H = 768
V = 32000
NH = 12
DH = H // NH
E = 8
FF = 1536
S = 2048
EPS = 1e-5


def _ln(x, g=None, b=None):
    m = jnp.mean(x, axis=-1, keepdims=True)
    v = jnp.mean(x * x, axis=-1, keepdims=True) - m * m
    y = (x - m) * jax.lax.rsqrt(v + EPS)
    if g is not None:
        y = y * g + b
    return y


def _dot(a, b):
    return jnp.dot(a.astype(jnp.bfloat16), b.astype(jnp.bfloat16),
                   preferred_element_type=jnp.float32)


def _pre_body(hs_ref, te_ref, pw1_ref, pw2_ref, pb_ref,
              n1g_ref, n1b_ref, ob_ref, xn_ref, y_ref):
    c0 = (_dot(_ln(hs_ref[...]), pw1_ref[...])
          + _dot(_ln(te_ref[...]), pw2_ref[...]) + pb_ref[...])
    xn_ref[...] = _ln(c0, n1g_ref[...], n1b_ref[...]).astype(jnp.bfloat16)
    y_ref[...] = c0 + ob_ref[...]


def _prologue(hs, te, pw1, pw2, pb, n1g, n1b, ob):
    return pl.pallas_call(
        _pre_body,
        out_shape=(jax.ShapeDtypeStruct((S, H), jnp.bfloat16),
                   jax.ShapeDtypeStruct((S, H), jnp.float32)),
    )(hs, te, pw1, pw2, pb, n1g, n1b, ob)


HPS = 4
AQC = 512


def _attn_body(xn_ref, y0_ref, wq_ref, wk_ref, wv_ref,
               bq_ref, bk_ref, bv_ref, wo_ref, out_ref):
    g = pl.program_id(0)
    xn = xn_ref[...]
    contribs = []
    for hh in range(HPS):
        q = _dot(xn, wq_ref[hh]) + bq_ref[hh]
        k = _dot(xn, wk_ref[hh]) + bk_ref[hh]
        v = (_dot(xn, wv_ref[hh]) + bv_ref[hh]).astype(jnp.bfloat16)
        kb = k.astype(jnp.bfloat16)
        wo_h = wo_ref[hh * DH:(hh + 1) * DH, :]
        for c in range(S // AQC):
            qc = q[c * AQC:(c + 1) * AQC, :]
            sc = jax.lax.dot_general(qc.astype(jnp.bfloat16), kb,
                                     (((1,), (1,)), ((), ())),
                                     preferred_element_type=jnp.float32)
            sc = sc * (1.0 / math.sqrt(DH))
            sc = sc - jnp.max(sc, axis=-1, keepdims=True)
            p = jnp.exp(sc.astype(jnp.bfloat16))
            r = jnp.sum(p.astype(jnp.float32), axis=-1, keepdims=True)
            o = jnp.dot(p, v, preferred_element_type=jnp.float32) / r
            contribs.append((c, _dot(o, wo_h)))
    acc = [jnp.zeros((AQC, H), jnp.float32) for _ in range(S // AQC)]
    for c, contrib in contribs:
        acc[c] = acc[c] + contrib
    for c in range(S // AQC):
        sl = slice(c * AQC, (c + 1) * AQC)

        @pl.when(g == 0)
        def _(sl=sl, c=c):
            out_ref[sl, :] = y0_ref[sl, :] + acc[c]

        @pl.when(g > 0)
        def _(sl=sl, c=c):
            out_ref[sl, :] += acc[c]


def _attention(xn, y0, qkv_Ws, qkv_bs, out_W):
    const = lambda h: (0, 0)
    specs = [
        pl.BlockSpec((S, H), const),
        pl.BlockSpec((S, H), const),
        pl.BlockSpec((HPS, H, DH), lambda g: (g, 0, 0)),
        pl.BlockSpec((HPS, H, DH), lambda g: (NH // HPS + g, 0, 0)),
        pl.BlockSpec((HPS, H, DH), lambda g: (2 * NH // HPS + g, 0, 0)),
        pl.BlockSpec((HPS, 1, DH), lambda g: (g, 0, 0)),
        pl.BlockSpec((HPS, 1, DH), lambda g: (NH // HPS + g, 0, 0)),
        pl.BlockSpec((HPS, 1, DH), lambda g: (2 * NH // HPS + g, 0, 0)),
        pl.BlockSpec((HPS * DH, H), lambda g: (g, 0)),
    ]
    return pl.pallas_call(
        _attn_body,
        grid=(NH // HPS,),
        in_specs=specs,
        out_specs=pl.BlockSpec((S, H), const),
        out_shape=jax.ShapeDtypeStruct((S, H), jnp.float32),
    )(xn, y0, qkv_Ws, qkv_Ws, qkv_Ws, qkv_bs, qkv_bs, qkv_bs, out_W)


def _moe_body(y_ref, g_ref, b_ref, gw_ref, gb_ref,
              w1_ref, b1_ref, w2_ref, b2_ref, out_ref, x2_s, wv_s):
    e = pl.program_id(0)

    @pl.when(e == 0)
    def _gate():
        x2 = _ln(y_ref[...], g_ref[...], b_ref[...])
        x2_s[...] = x2.astype(jnp.bfloat16)
        logits = _dot(x2, gw_ref[...]) + gb_ref[...]
        lane = jax.lax.broadcasted_iota(jnp.int32, logits.shape, 1)
        logits = jnp.where(lane < E, logits, -1e30)
        logits = logits - jnp.max(logits, axis=-1, keepdims=True)
        pexp = jnp.exp(logits)
        probs = pexp / jnp.sum(pexp, axis=-1, keepdims=True)
        m1 = jnp.max(probs, axis=-1, keepdims=True)
        m2 = jnp.max(jnp.where(probs == m1, -1.0, probs),
                     axis=-1, keepdims=True)
        wv_s[...] = jnp.where(probs >= m2, probs, 0.0) / (m1 + m2)

    x2 = x2_s[...]
    lane = jax.lax.broadcasted_iota(jnp.int32, (S, 128), 1)
    onehot = (lane == e).astype(jnp.float32)
    we = jnp.sum(wv_s[...] * onehot, axis=-1, keepdims=True)
    hmat = jnp.maximum(
        jnp.dot(x2, w1_ref[0].astype(jnp.bfloat16),
                preferred_element_type=jnp.float32) + b1_ref[0],
        0.0).astype(jnp.bfloat16)
    contrib = (jnp.dot(hmat, w2_ref[0].astype(jnp.bfloat16),
                       preferred_element_type=jnp.float32)
               + b2_ref[0]) * we

    @pl.when(e == 0)
    def _():
        out_ref[...] = y_ref[...] + contrib

    @pl.when(e > 0)
    def _():
        out_ref[...] += contrib


def _moe(y, n2g, n2b, gw_pad, gb_pad, w1, b1, w2, b2):
    const = lambda e: (0, 0)
    specs = [
        pl.BlockSpec((S, H), const),
        pl.BlockSpec((1, H), const),
        pl.BlockSpec((1, H), const),
        pl.BlockSpec((H, 128), const),
        pl.BlockSpec((1, 128), const),
        pl.BlockSpec((1, H, FF), lambda e: (e, 0, 0)),
        pl.BlockSpec((1, 1, FF), lambda e: (e, 0, 0)),
        pl.BlockSpec((1, FF, H), lambda e: (e, 0, 0)),
        pl.BlockSpec((1, 1, H), lambda e: (e, 0, 0)),
    ]
    return pl.pallas_call(
        _moe_body,
        grid=(E,),
        in_specs=specs,
        out_specs=pl.BlockSpec((S, H), const),
        out_shape=jax.ShapeDtypeStruct((S, H), jnp.float32),
        scratch_shapes=[
            pltpu.VMEM((S, H), jnp.bfloat16),
            pltpu.VMEM((S, 128), jnp.float32),
        ],
    )(y, n2g, n2b, gw_pad, gb_pad, w1, b1, w2, b2)


VB = 1280
NVB = V // VB


def _head_body(x_ref, w_ref, b_ref, out_hbm, buf, sems):
    j = pl.program_id(0)
    slot = j % 2

    @pl.when(j >= 2)
    def _():
        pltpu.make_async_copy(
            buf.at[slot], out_hbm.at[:, pl.ds((j - 2) * VB, VB)],
            sems.at[slot]).wait()

    buf[slot] = _dot(x_ref[...], w_ref[...]) + b_ref[...]
    pltpu.make_async_copy(
        buf.at[slot], out_hbm.at[:, pl.ds(j * VB, VB)], sems.at[slot]).start()

    @pl.when(j == NVB - 1)
    def _():
        pltpu.make_async_copy(
            buf.at[1 - slot], out_hbm.at[:, pl.ds((j - 1) * VB, VB)],
            sems.at[1 - slot]).wait()
        pltpu.make_async_copy(
            buf.at[slot], out_hbm.at[:, pl.ds(j * VB, VB)],
            sems.at[slot]).wait()


def _head(x, hw, hb):
    return pl.pallas_call(
        _head_body,
        grid=(NVB,),
        in_specs=[
            pl.BlockSpec((S, H), lambda j: (0, 0)),
            pl.BlockSpec((H, VB), lambda j: (0, j)),
            pl.BlockSpec((1, VB), lambda j: (0, j)),
        ],
        out_specs=pl.BlockSpec(memory_space=pl.ANY),
        out_shape=jax.ShapeDtypeStruct((S, V), jnp.float32),
        scratch_shapes=[
            pltpu.VMEM((2, S, VB), jnp.float32),
            pltpu.SemaphoreType.DMA((2,)),
        ],
    )(x, hw, hb)


def kernel(hidden_states, token_embeddings, proj_W, proj_b, qkv_W, qkv_b,
           attn_out_W, attn_out_b, norm1_g, norm1_b, norm2_g, norm2_b,
           gate_W, gate_b, w1, b1, w2, b2, head_W, head_b):
    nmtp = proj_W.shape[0]
    hs = hidden_states.reshape(S, H)
    outs = []
    for i in range(nmtp):
        gw_pad = jnp.pad(gate_W[i], ((0, 0), (0, 128 - E)))
        gb_pad = jnp.pad(gate_b[i], (0, 128 - E)).reshape(1, 128)
        qkv_Ws = qkv_W[i].reshape(H, 3 * NH, DH).transpose(1, 0, 2)
        qkv_bs = qkv_b[i].reshape(3 * NH, 1, DH)
        xn, y0 = _prologue(hs, token_embeddings[i, 0],
                           proj_W[i, :H], proj_W[i, H:],
                           proj_b[i].reshape(1, H),
                           norm1_g[i].reshape(1, H), norm1_b[i].reshape(1, H),
                           attn_out_b[i].reshape(1, H))
        y = _attention(xn, y0, qkv_Ws, qkv_bs, attn_out_W[i])
        z = _moe(y, norm2_g[i].reshape(1, H), norm2_b[i].reshape(1, H),
                 gw_pad, gb_pad, w1[i], b1[i].reshape(E, 1, FF),
                 w2[i], b2[i].reshape(E, 1, H))
        outs.append(_head(z, head_W[i], head_b[i].reshape(1, V)))
    mtp_logits = jnp.stack(outs)[:, None]
    return mtp_logits, jnp.zeros((), jnp.float32)


# speedup vs baseline: 1.0537x; 1.0537x over previous
import math

import jax
import jax.numpy as jnp
from jax.experimental import pallas as pl
import jax.experimental.pallas.tpu as pltpu

H = 768
V = 32000
NH = 12
DH = H // NH
E = 8
FF = 1536
S = 2048
EPS = 1e-5


def _ln(x, g=None, b=None):
    m = jnp.mean(x, axis=-1, keepdims=True)
    v = jnp.mean(x * x, axis=-1, keepdims=True) - m * m
    y = (x - m) * jax.lax.rsqrt(v + EPS)
    if g is not None:
        y = y * g + b
    return y


def _dot(a, b):
    return jnp.dot(a.astype(jnp.bfloat16), b.astype(jnp.bfloat16),
                   preferred_element_type=jnp.float32)


def _pre_body(hs_ref, te_ref, pw1_ref, pw2_ref, pb_ref,
              n1g_ref, n1b_ref, ob_ref, xn_ref, y_ref):
    c0 = (_dot(_ln(hs_ref[...]), pw1_ref[...])
          + _dot(_ln(te_ref[...]), pw2_ref[...]) + pb_ref[...])
    xn_ref[...] = _ln(c0, n1g_ref[...], n1b_ref[...]).astype(jnp.bfloat16)
    y_ref[...] = c0 + ob_ref[...]


def _prologue(hs, te, pw1, pw2, pb, n1g, n1b, ob):
    return pl.pallas_call(
        _pre_body,
        out_shape=(jax.ShapeDtypeStruct((S, H), jnp.bfloat16),
                   jax.ShapeDtypeStruct((S, H), jnp.float32)),
    )(hs, te, pw1, pw2, pb, n1g, n1b, ob)


HPS = 3
AQC = 1024


def _attn_body(xn_ref, y0_ref, wq_ref, wk_ref, wv_ref,
               bq_ref, bk_ref, bv_ref, wo_ref, out_ref):
    g = pl.program_id(0)
    xn = xn_ref[...]
    contribs = []
    for hh in range(HPS):
        q = _dot(xn, wq_ref[hh]) + bq_ref[hh]
        k = _dot(xn, wk_ref[hh]) + bk_ref[hh]
        v = (_dot(xn, wv_ref[hh]) + bv_ref[hh]).astype(jnp.bfloat16)
        kb = k.astype(jnp.bfloat16)
        wo_h = wo_ref[hh * DH:(hh + 1) * DH, :]
        for c in range(S // AQC):
            qc = q[c * AQC:(c + 1) * AQC, :]
            sc = jax.lax.dot_general(qc.astype(jnp.bfloat16), kb,
                                     (((1,), (1,)), ((), ())),
                                     preferred_element_type=jnp.float32)
            sc = sc * (1.0 / math.sqrt(DH))
            sc = sc - jnp.max(sc, axis=-1, keepdims=True)
            p = jnp.exp(sc.astype(jnp.bfloat16))
            r = jnp.sum(p.astype(jnp.float32), axis=-1, keepdims=True)
            o = jnp.dot(p, v, preferred_element_type=jnp.float32) / r
            contribs.append((c, _dot(o, wo_h)))
    acc = [jnp.zeros((AQC, H), jnp.float32) for _ in range(S // AQC)]
    for c, contrib in contribs:
        acc[c] = acc[c] + contrib
    for c in range(S // AQC):
        sl = slice(c * AQC, (c + 1) * AQC)

        @pl.when(g == 0)
        def _(sl=sl, c=c):
            out_ref[sl, :] = y0_ref[sl, :] + acc[c]

        @pl.when(g > 0)
        def _(sl=sl, c=c):
            out_ref[sl, :] += acc[c]


def _attention(xn, y0, qkv_Ws, qkv_bs, out_W):
    const = lambda h: (0, 0)
    specs = [
        pl.BlockSpec((S, H), const),
        pl.BlockSpec((S, H), const),
        pl.BlockSpec((HPS, H, DH), lambda g: (g, 0, 0)),
        pl.BlockSpec((HPS, H, DH), lambda g: (NH // HPS + g, 0, 0)),
        pl.BlockSpec((HPS, H, DH), lambda g: (2 * NH // HPS + g, 0, 0)),
        pl.BlockSpec((HPS, 1, DH), lambda g: (g, 0, 0)),
        pl.BlockSpec((HPS, 1, DH), lambda g: (NH // HPS + g, 0, 0)),
        pl.BlockSpec((HPS, 1, DH), lambda g: (2 * NH // HPS + g, 0, 0)),
        pl.BlockSpec((HPS * DH, H), lambda g: (g, 0)),
    ]
    return pl.pallas_call(
        _attn_body,
        grid=(NH // HPS,),
        in_specs=specs,
        out_specs=pl.BlockSpec((S, H), const),
        out_shape=jax.ShapeDtypeStruct((S, H), jnp.float32),
    )(xn, y0, qkv_Ws, qkv_Ws, qkv_Ws, qkv_bs, qkv_bs, qkv_bs, out_W)


def _moe_body(y_ref, g_ref, b_ref, gw_ref, gb_ref,
              w1_ref, b1_ref, w2_ref, b2_ref, out_ref, x2_s, wv_s):
    e = pl.program_id(0)

    @pl.when(e == 0)
    def _gate():
        x2 = _ln(y_ref[...], g_ref[...], b_ref[...])
        x2_s[...] = x2.astype(jnp.bfloat16)
        logits = _dot(x2, gw_ref[...]) + gb_ref[...]
        lane = jax.lax.broadcasted_iota(jnp.int32, logits.shape, 1)
        logits = jnp.where(lane < E, logits, -1e30)
        logits = logits - jnp.max(logits, axis=-1, keepdims=True)
        pexp = jnp.exp(logits)
        probs = pexp / jnp.sum(pexp, axis=-1, keepdims=True)
        m1 = jnp.max(probs, axis=-1, keepdims=True)
        m2 = jnp.max(jnp.where(probs == m1, -1.0, probs),
                     axis=-1, keepdims=True)
        wv_s[...] = jnp.where(probs >= m2, probs, 0.0) / (m1 + m2)

    x2 = x2_s[...]
    lane = jax.lax.broadcasted_iota(jnp.int32, (S, 128), 1)
    onehot = (lane == e).astype(jnp.float32)
    we = jnp.sum(wv_s[...] * onehot, axis=-1, keepdims=True)
    hmat = jnp.maximum(
        jnp.dot(x2, w1_ref[0].astype(jnp.bfloat16),
                preferred_element_type=jnp.float32) + b1_ref[0],
        0.0).astype(jnp.bfloat16)
    contrib = (jnp.dot(hmat, w2_ref[0].astype(jnp.bfloat16),
                       preferred_element_type=jnp.float32)
               + b2_ref[0]) * we

    @pl.when(e == 0)
    def _():
        out_ref[...] = y_ref[...] + contrib

    @pl.when(e > 0)
    def _():
        out_ref[...] += contrib


def _moe(y, n2g, n2b, gw_pad, gb_pad, w1, b1, w2, b2):
    const = lambda e: (0, 0)
    specs = [
        pl.BlockSpec((S, H), const),
        pl.BlockSpec((1, H), const),
        pl.BlockSpec((1, H), const),
        pl.BlockSpec((H, 128), const),
        pl.BlockSpec((1, 128), const),
        pl.BlockSpec((1, H, FF), lambda e: (e, 0, 0)),
        pl.BlockSpec((1, 1, FF), lambda e: (e, 0, 0)),
        pl.BlockSpec((1, FF, H), lambda e: (e, 0, 0)),
        pl.BlockSpec((1, 1, H), lambda e: (e, 0, 0)),
    ]
    return pl.pallas_call(
        _moe_body,
        grid=(E,),
        in_specs=specs,
        out_specs=pl.BlockSpec((S, H), const),
        out_shape=jax.ShapeDtypeStruct((S, H), jnp.float32),
        scratch_shapes=[
            pltpu.VMEM((S, H), jnp.bfloat16),
            pltpu.VMEM((S, 128), jnp.float32),
        ],
    )(y, n2g, n2b, gw_pad, gb_pad, w1, b1, w2, b2)


VB = 1280
NVB = V // VB


def _head_body(x_ref, w_ref, b_ref, out_hbm, buf, sems):
    j = pl.program_id(0)
    slot = j % 2

    @pl.when(j >= 2)
    def _():
        pltpu.make_async_copy(
            buf.at[slot], out_hbm.at[:, pl.ds((j - 2) * VB, VB)],
            sems.at[slot]).wait()

    buf[slot] = _dot(x_ref[...], w_ref[...]) + b_ref[...]
    pltpu.make_async_copy(
        buf.at[slot], out_hbm.at[:, pl.ds(j * VB, VB)], sems.at[slot]).start()

    @pl.when(j == NVB - 1)
    def _():
        pltpu.make_async_copy(
            buf.at[1 - slot], out_hbm.at[:, pl.ds((j - 1) * VB, VB)],
            sems.at[1 - slot]).wait()
        pltpu.make_async_copy(
            buf.at[slot], out_hbm.at[:, pl.ds(j * VB, VB)],
            sems.at[slot]).wait()


def _head(x, hw, hb):
    return pl.pallas_call(
        _head_body,
        grid=(NVB,),
        in_specs=[
            pl.BlockSpec((S, H), lambda j: (0, 0)),
            pl.BlockSpec((H, VB), lambda j: (0, j)),
            pl.BlockSpec((1, VB), lambda j: (0, j)),
        ],
        out_specs=pl.BlockSpec(memory_space=pl.ANY),
        out_shape=jax.ShapeDtypeStruct((S, V), jnp.float32),
        scratch_shapes=[
            pltpu.VMEM((2, S, VB), jnp.float32),
            pltpu.SemaphoreType.DMA((2,)),
        ],
    )(x, hw, hb)


def kernel(hidden_states, token_embeddings, proj_W, proj_b, qkv_W, qkv_b,
           attn_out_W, attn_out_b, norm1_g, norm1_b, norm2_g, norm2_b,
           gate_W, gate_b, w1, b1, w2, b2, head_W, head_b):
    nmtp = proj_W.shape[0]
    hs = hidden_states.reshape(S, H)
    outs = []
    for i in range(nmtp):
        gw_pad = jnp.pad(gate_W[i], ((0, 0), (0, 128 - E)))
        gb_pad = jnp.pad(gate_b[i], (0, 128 - E)).reshape(1, 128)
        qkv_Ws = qkv_W[i].reshape(H, 3 * NH, DH).transpose(1, 0, 2)
        qkv_bs = qkv_b[i].reshape(3 * NH, 1, DH)
        xn, y0 = _prologue(hs, token_embeddings[i, 0],
                           proj_W[i, :H], proj_W[i, H:],
                           proj_b[i].reshape(1, H),
                           norm1_g[i].reshape(1, H), norm1_b[i].reshape(1, H),
                           attn_out_b[i].reshape(1, H))
        y = _attention(xn, y0, qkv_Ws, qkv_bs, attn_out_W[i])
        z = _moe(y, norm2_g[i].reshape(1, H), norm2_b[i].reshape(1, H),
                 gw_pad, gb_pad, w1[i], b1[i].reshape(E, 1, FF),
                 w2[i], b2[i].reshape(E, 1, H))
        outs.append(_head(z, head_W[i], head_b[i].reshape(1, V)))
    mtp_logits = jnp.stack(outs)[:, None]
    return mtp_logits, jnp.zeros((), jnp.float32)


# speedup vs baseline: 1.1006x; 1.0445x over previous
import math

import jax
import jax.numpy as jnp
from jax.experimental import pallas as pl
import jax.experimental.pallas.tpu as pltpu

H = 768
V = 32000
NH = 12
DH = H // NH
E = 8
FF = 1536
S = 2048
EPS = 1e-5


def _ln(x, g=None, b=None):
    m = jnp.mean(x, axis=-1, keepdims=True)
    v = jnp.mean(x * x, axis=-1, keepdims=True) - m * m
    y = (x - m) * jax.lax.rsqrt(v + EPS)
    if g is not None:
        y = y * g + b
    return y


def _dot(a, b):
    return jnp.dot(a.astype(jnp.bfloat16), b.astype(jnp.bfloat16),
                   preferred_element_type=jnp.float32)


def _pre_body(hs_ref, te_ref, pw1_ref, pw2_ref, pb_ref,
              n1g_ref, n1b_ref, ob_ref, xn_ref, y_ref):
    c0 = (_dot(_ln(hs_ref[...]), pw1_ref[...])
          + _dot(_ln(te_ref[...]), pw2_ref[...]) + pb_ref[...])
    xn_ref[...] = _ln(c0, n1g_ref[...], n1b_ref[...]).astype(jnp.bfloat16)
    y_ref[...] = c0 + ob_ref[...]


def _prologue(hs, te, pw1, pw2, pb, n1g, n1b, ob):
    return pl.pallas_call(
        _pre_body,
        out_shape=(jax.ShapeDtypeStruct((S, H), jnp.bfloat16),
                   jax.ShapeDtypeStruct((S, H), jnp.float32)),
    )(hs, te, pw1, pw2, pb, n1g, n1b, ob)


HPS = 2
AQC = 1024


def _attn_body(xn_ref, y0_ref, wq_ref, wk_ref, wv_ref,
               bq_ref, bk_ref, bv_ref, wo_ref, out_ref):
    g = pl.program_id(0)
    xn = xn_ref[...]
    contribs = []
    for hh in range(HPS):
        q = _dot(xn, wq_ref[hh]) + bq_ref[hh]
        k = _dot(xn, wk_ref[hh]) + bk_ref[hh]
        v = (_dot(xn, wv_ref[hh]) + bv_ref[hh]).astype(jnp.bfloat16)
        kb = k.astype(jnp.bfloat16)
        wo_h = wo_ref[hh * DH:(hh + 1) * DH, :]
        for c in range(S // AQC):
            qc = q[c * AQC:(c + 1) * AQC, :]
            sc = jax.lax.dot_general(qc.astype(jnp.bfloat16), kb,
                                     (((1,), (1,)), ((), ())),
                                     preferred_element_type=jnp.float32)
            sc = sc * (1.0 / math.sqrt(DH))
            sc = sc - jnp.max(sc, axis=-1, keepdims=True)
            p = jnp.exp(sc.astype(jnp.bfloat16))
            r = jnp.sum(p.astype(jnp.float32), axis=-1, keepdims=True)
            o = jnp.dot(p, v, preferred_element_type=jnp.float32) / r
            contribs.append((c, _dot(o, wo_h)))
    acc = [jnp.zeros((AQC, H), jnp.float32) for _ in range(S // AQC)]
    for c, contrib in contribs:
        acc[c] = acc[c] + contrib
    for c in range(S // AQC):
        sl = slice(c * AQC, (c + 1) * AQC)

        @pl.when(g == 0)
        def _(sl=sl, c=c):
            out_ref[sl, :] = y0_ref[sl, :] + acc[c]

        @pl.when(g > 0)
        def _(sl=sl, c=c):
            out_ref[sl, :] += acc[c]


def _attention(xn, y0, qkv_Ws, qkv_bs, out_W):
    const = lambda h: (0, 0)
    specs = [
        pl.BlockSpec((S, H), const),
        pl.BlockSpec((S, H), const),
        pl.BlockSpec((HPS, H, DH), lambda g: (g, 0, 0)),
        pl.BlockSpec((HPS, H, DH), lambda g: (NH // HPS + g, 0, 0)),
        pl.BlockSpec((HPS, H, DH), lambda g: (2 * NH // HPS + g, 0, 0)),
        pl.BlockSpec((HPS, 1, DH), lambda g: (g, 0, 0)),
        pl.BlockSpec((HPS, 1, DH), lambda g: (NH // HPS + g, 0, 0)),
        pl.BlockSpec((HPS, 1, DH), lambda g: (2 * NH // HPS + g, 0, 0)),
        pl.BlockSpec((HPS * DH, H), lambda g: (g, 0)),
    ]
    return pl.pallas_call(
        _attn_body,
        grid=(NH // HPS,),
        in_specs=specs,
        out_specs=pl.BlockSpec((S, H), const),
        out_shape=jax.ShapeDtypeStruct((S, H), jnp.float32),
    )(xn, y0, qkv_Ws, qkv_Ws, qkv_Ws, qkv_bs, qkv_bs, qkv_bs, out_W)


def _moe_body(y_ref, g_ref, b_ref, gw_ref, gb_ref,
              w1_ref, b1_ref, w2_ref, b2_ref, out_ref, x2_s, wv_s):
    e = pl.program_id(0)

    @pl.when(e == 0)
    def _gate():
        x2 = _ln(y_ref[...], g_ref[...], b_ref[...])
        x2_s[...] = x2.astype(jnp.bfloat16)
        logits = _dot(x2, gw_ref[...]) + gb_ref[...]
        lane = jax.lax.broadcasted_iota(jnp.int32, logits.shape, 1)
        logits = jnp.where(lane < E, logits, -1e30)
        logits = logits - jnp.max(logits, axis=-1, keepdims=True)
        pexp = jnp.exp(logits)
        probs = pexp / jnp.sum(pexp, axis=-1, keepdims=True)
        m1 = jnp.max(probs, axis=-1, keepdims=True)
        m2 = jnp.max(jnp.where(probs == m1, -1.0, probs),
                     axis=-1, keepdims=True)
        wv_s[...] = jnp.where(probs >= m2, probs, 0.0) / (m1 + m2)

    x2 = x2_s[...]
    lane = jax.lax.broadcasted_iota(jnp.int32, (S, 128), 1)
    onehot = (lane == e).astype(jnp.float32)
    we = jnp.sum(wv_s[...] * onehot, axis=-1, keepdims=True)
    hmat = jnp.maximum(
        jnp.dot(x2, w1_ref[0].astype(jnp.bfloat16),
                preferred_element_type=jnp.float32) + b1_ref[0],
        0.0).astype(jnp.bfloat16)
    contrib = (jnp.dot(hmat, w2_ref[0].astype(jnp.bfloat16),
                       preferred_element_type=jnp.float32)
               + b2_ref[0]) * we

    @pl.when(e == 0)
    def _():
        out_ref[...] = y_ref[...] + contrib

    @pl.when(e > 0)
    def _():
        out_ref[...] += contrib


def _moe(y, n2g, n2b, gw_pad, gb_pad, w1, b1, w2, b2):
    const = lambda e: (0, 0)
    specs = [
        pl.BlockSpec((S, H), const),
        pl.BlockSpec((1, H), const),
        pl.BlockSpec((1, H), const),
        pl.BlockSpec((H, 128), const),
        pl.BlockSpec((1, 128), const),
        pl.BlockSpec((1, H, FF), lambda e: (e, 0, 0)),
        pl.BlockSpec((1, 1, FF), lambda e: (e, 0, 0)),
        pl.BlockSpec((1, FF, H), lambda e: (e, 0, 0)),
        pl.BlockSpec((1, 1, H), lambda e: (e, 0, 0)),
    ]
    return pl.pallas_call(
        _moe_body,
        grid=(E,),
        in_specs=specs,
        out_specs=pl.BlockSpec((S, H), const),
        out_shape=jax.ShapeDtypeStruct((S, H), jnp.float32),
        scratch_shapes=[
            pltpu.VMEM((S, H), jnp.bfloat16),
            pltpu.VMEM((S, 128), jnp.float32),
        ],
    )(y, n2g, n2b, gw_pad, gb_pad, w1, b1, w2, b2)


VB = 1280
NVB = V // VB


def _head_body(x_ref, w_ref, b_ref, out_hbm, buf, sems):
    j = pl.program_id(0)
    slot = j % 2

    @pl.when(j >= 2)
    def _():
        pltpu.make_async_copy(
            buf.at[slot], out_hbm.at[:, pl.ds((j - 2) * VB, VB)],
            sems.at[slot]).wait()

    buf[slot] = _dot(x_ref[...], w_ref[...]) + b_ref[...]
    pltpu.make_async_copy(
        buf.at[slot], out_hbm.at[:, pl.ds(j * VB, VB)], sems.at[slot]).start()

    @pl.when(j == NVB - 1)
    def _():
        pltpu.make_async_copy(
            buf.at[1 - slot], out_hbm.at[:, pl.ds((j - 1) * VB, VB)],
            sems.at[1 - slot]).wait()
        pltpu.make_async_copy(
            buf.at[slot], out_hbm.at[:, pl.ds(j * VB, VB)],
            sems.at[slot]).wait()


def _head(x, hw, hb):
    return pl.pallas_call(
        _head_body,
        grid=(NVB,),
        in_specs=[
            pl.BlockSpec((S, H), lambda j: (0, 0)),
            pl.BlockSpec((H, VB), lambda j: (0, j)),
            pl.BlockSpec((1, VB), lambda j: (0, j)),
        ],
        out_specs=pl.BlockSpec(memory_space=pl.ANY),
        out_shape=jax.ShapeDtypeStruct((S, V), jnp.float32),
        scratch_shapes=[
            pltpu.VMEM((2, S, VB), jnp.float32),
            pltpu.SemaphoreType.DMA((2,)),
        ],
    )(x, hw, hb)


def kernel(hidden_states, token_embeddings, proj_W, proj_b, qkv_W, qkv_b,
           attn_out_W, attn_out_b, norm1_g, norm1_b, norm2_g, norm2_b,
           gate_W, gate_b, w1, b1, w2, b2, head_W, head_b):
    nmtp = proj_W.shape[0]
    hs = hidden_states.reshape(S, H)
    outs = []
    for i in range(nmtp):
        gw_pad = jnp.pad(gate_W[i], ((0, 0), (0, 128 - E)))
        gb_pad = jnp.pad(gate_b[i], (0, 128 - E)).reshape(1, 128)
        qkv_Ws = qkv_W[i].reshape(H, 3 * NH, DH).transpose(1, 0, 2)
        qkv_bs = qkv_b[i].reshape(3 * NH, 1, DH)
        xn, y0 = _prologue(hs, token_embeddings[i, 0],
                           proj_W[i, :H], proj_W[i, H:],
                           proj_b[i].reshape(1, H),
                           norm1_g[i].reshape(1, H), norm1_b[i].reshape(1, H),
                           attn_out_b[i].reshape(1, H))
        y = _attention(xn, y0, qkv_Ws, qkv_bs, attn_out_W[i])
        z = _moe(y, norm2_g[i].reshape(1, H), norm2_b[i].reshape(1, H),
                 gw_pad, gb_pad, w1[i], b1[i].reshape(E, 1, FF),
                 w2[i], b2[i].reshape(E, 1, H))
        outs.append(_head(z, head_W[i], head_b[i].reshape(1, V)))
    mtp_logits = jnp.stack(outs)[:, None]
    return mtp_logits, jnp.zeros((), jnp.float32)
